# Initial kernel scaffold; baseline (speedup 1.0000x reference)
import jax
import jax.numpy as jnp
from jax.experimental import pallas as pl


def kernel(x, token_table, pos_table):
    raise NotImplementedError("write your pallas kernel here")


# speedup vs baseline: 1.0535x; 1.0535x over previous
import functools

import jax
import jax.numpy as jnp
from jax import lax
from jax.experimental import pallas as pl
from jax.experimental.pallas import tpu as pltpu
from jax.experimental.pallas import tpu_sc as plsc

VOCAB = 1000000
MAXLEN = 200
EDIM = 32
BATCH = 4096

TOTAL_ROWS = BATCH * MAXLEN
UNIT = 128


def _build_sc_kernel():
    info = plsc.get_sparse_core_info()
    nc, ns = info.num_cores, info.num_subcores
    nw = nc * ns
    rows_per_w = TOTAL_ROWS // nw
    units = rows_per_w // UNIT

    mesh = plsc.VectorSubcoreMesh(core_axis_name="c", subcore_axis_name="s")

    @functools.partial(
        pl.kernel,
        mesh=mesh,
        out_type=jax.ShapeDtypeStruct((TOTAL_ROWS, EDIM), jnp.float32),
        scratch_types=[
            pltpu.VMEM((rows_per_w,), jnp.int32),
            pltpu.VMEM((UNIT, EDIM), jnp.float32),
            pltpu.VMEM((2 * MAXLEN, EDIM), jnp.float32),
            pltpu.SemaphoreType.DMA,
        ],
        compiler_params=pltpu.CompilerParams(use_tc_tiling_on_sc=False),
    )
    def sc_kernel(table_hbm, idx_hbm, pos_hbm, out_hbm, idx_v, rows_v, pos_v, sem):
        wid = lax.axis_index("s") * nc + lax.axis_index("c")
        base = wid * rows_per_w
        pltpu.sync_copy(idx_hbm.at[pl.ds(base, rows_per_w)], idx_v)
        pltpu.sync_copy(pos_hbm, pos_v)

        def unit_body(u, carry):
            pltpu.async_copy(
                table_hbm.at[idx_v.at[pl.ds(u * UNIT, UNIT)]], rows_v, sem
            ).wait()
            p0 = lax.rem(u * UNIT, MAXLEN)

            def row_add(r, c):
                p = p0 + r
                for h in range(EDIM // 16):
                    sl = pl.ds(h * 16, 16)
                    rows_v[r, sl] = rows_v[r, sl] + pos_v[p, sl]
                return c

            lax.fori_loop(0, UNIT, row_add, 0)
            pltpu.sync_copy(rows_v, out_hbm.at[pl.ds(base + u * UNIT, UNIT)])
            return carry

        lax.fori_loop(0, units, unit_body, 0)

    return sc_kernel


_SC_KERNEL = _build_sc_kernel()


@jax.jit
def kernel(x, token_table, pos_table):
    xf = x.reshape(-1).astype(jnp.int32)
    pos_dup = jnp.concatenate([pos_table, pos_table], axis=0)
    out = _SC_KERNEL(token_table, xf, pos_dup)
    return out.reshape(BATCH, MAXLEN, EDIM)


# speedup vs baseline: 1.1290x; 1.0717x over previous
import functools
import math

import jax
import jax.numpy as jnp
from jax import lax
from jax.experimental import pallas as pl
from jax.experimental.pallas import tpu as pltpu
from jax.experimental.pallas import tpu_sc as plsc

VOCAB = 1000000
MAXLEN = 200
EDIM = 32
BATCH = 4096

TOTAL_ROWS = BATCH * MAXLEN
UNIT = 128


def _build_sc_kernel():
    info = plsc.get_sparse_core_info()
    nc, ns = info.num_cores, info.num_subcores
    nw = nc * ns
    rows_per_w = TOTAL_ROWS // nw
    units = rows_per_w // UNIT

    mesh = plsc.VectorSubcoreMesh(core_axis_name="c", subcore_axis_name="s")
    phases = (UNIT * MAXLEN // math.gcd(UNIT, MAXLEN)) // UNIT

    @functools.partial(
        pl.kernel,
        mesh=mesh,
        out_type=jax.ShapeDtypeStruct((TOTAL_ROWS, EDIM), jnp.float32),
        scratch_types=[
            pltpu.VMEM((rows_per_w,), jnp.int32),
            pltpu.VMEM((UNIT, EDIM), jnp.float32),
            pltpu.SemaphoreType.DMA,
        ],
        compiler_params=pltpu.CompilerParams(use_tc_tiling_on_sc=False),
    )
    def sc_kernel(table_hbm, idx_hbm, pos_hbm, out_hbm, idx_v, rows_v, sem):
        wid = lax.axis_index("s") * nc + lax.axis_index("c")
        base = wid * rows_per_w
        pltpu.sync_copy(idx_hbm.at[pl.ds(base, rows_per_w)], idx_v)

        def unit_body(u, carry):
            phase = lax.rem(u, phases)
            pltpu.sync_copy(pos_hbm.at[pl.ds(phase * UNIT, UNIT)], rows_v)
            pltpu.async_copy(
                table_hbm.at[idx_v.at[pl.ds(u * UNIT, UNIT)]], rows_v, sem,
                add=True,
            ).wait()
            pltpu.sync_copy(rows_v, out_hbm.at[pl.ds(base + u * UNIT, UNIT)])
            return carry

        lax.fori_loop(0, units, unit_body, 0)

    return sc_kernel


_SC_KERNEL = _build_sc_kernel()


@jax.jit
def kernel(x, token_table, pos_table):
    xf = x.reshape(-1).astype(jnp.int32)
    reps = (UNIT * MAXLEN // math.gcd(UNIT, MAXLEN)) // MAXLEN
    pos_pat = jnp.tile(pos_table, (reps, 1))
    out = _SC_KERNEL(token_table, xf, pos_pat)
    return out.reshape(BATCH, MAXLEN, EDIM)


# speedup vs baseline: 1.3933x; 1.2341x over previous
import functools
import math

import jax
import jax.numpy as jnp
from jax import lax
from jax.experimental import pallas as pl
from jax.experimental.pallas import tpu as pltpu
from jax.experimental.pallas import tpu_sc as plsc

VOCAB = 1000000
MAXLEN = 200
EDIM = 32
BATCH = 4096

TOTAL_ROWS = BATCH * MAXLEN
GATHER = 128
CHUNK = 2
UROWS = GATHER * CHUNK
NBUF = 4

_LCM = UROWS * MAXLEN // math.gcd(UROWS, MAXLEN)
PHASES = _LCM // UROWS


def _build_sc_kernel():
    info = plsc.get_sparse_core_info()
    nc, ns = info.num_cores, info.num_subcores
    nw = nc * ns
    rows_per_w = TOTAL_ROWS // nw
    units = rows_per_w // UROWS
    padded_steps = ((units + 2 + NBUF - 1) // NBUF) * NBUF

    mesh = plsc.VectorSubcoreMesh(core_axis_name="c", subcore_axis_name="s")

    @functools.partial(
        pl.kernel,
        mesh=mesh,
        out_type=jax.ShapeDtypeStruct((TOTAL_ROWS, EDIM), jnp.float32),
        scratch_types=(
            [pltpu.VMEM((rows_per_w,), jnp.int32)]
            + [pltpu.VMEM((UROWS, EDIM), jnp.float32) for _ in range(NBUF)]
            + [pltpu.SemaphoreType.DMA for _ in range(3 * NBUF)]
        ),
        compiler_params=pltpu.CompilerParams(use_tc_tiling_on_sc=False),
    )
    def sc_kernel(table_hbm, idx_hbm, pos_hbm, out_hbm, idx_v, *rest):
        bufs = rest[:NBUF]
        psem = rest[NBUF:2 * NBUF]
        gsem = rest[2 * NBUF:3 * NBUF]
        ssem = rest[3 * NBUF:4 * NBUF]

        wid = lax.axis_index("s") * nc + lax.axis_index("c")
        base = wid * rows_per_w
        pltpu.sync_copy(idx_hbm.at[pl.ds(base, rows_per_w)], idx_v)

        def start_p(u, b):
            ph = lax.rem(u, PHASES)
            pltpu.async_copy(pos_hbm.at[pl.ds(ph * UROWS, UROWS)], bufs[b], psem[b])

        def wait_p(b):
            pltpu.make_async_copy(
                pos_hbm.at[pl.ds(0, UROWS)], bufs[b], psem[b]).wait()

        def start_g(u, b):
            for j in range(CHUNK):
                pltpu.async_copy(
                    table_hbm.at[idx_v.at[pl.ds(u * UROWS + j * GATHER, GATHER)]],
                    bufs[b].at[pl.ds(j * GATHER, GATHER)], gsem[b], add=True)

        def wait_g(b):
            for j in range(CHUNK):
                pltpu.make_async_copy(
                    table_hbm.at[idx_v.at[pl.ds(j * GATHER, GATHER)]],
                    bufs[b].at[pl.ds(j * GATHER, GATHER)], gsem[b]).wait()

        def start_s(u, b):
            pltpu.async_copy(
                bufs[b], out_hbm.at[pl.ds(base + u * UROWS, UROWS)], ssem[b])

        def wait_s(b):
            pltpu.make_async_copy(
                bufs[b], out_hbm.at[pl.ds(base, UROWS)], ssem[b]).wait()

        @pl.loop(0, padded_steps, step=NBUF)
        def step_loop(s):
            for b in range(NBUF):
                u = s + b

                @pl.when(jnp.logical_and(u >= NBUF, u < units))
                def _():
                    wait_s(b)

                @pl.when(u < units)
                def _():
                    start_p(u, b)

                b1 = (b - 1) % NBUF
                u1 = u - 1

                @pl.when(jnp.logical_and(u1 >= 0, u1 < units))
                def _():
                    wait_p(b1)
                    start_g(u1, b1)

                b2 = (b - 2) % NBUF
                u2 = u - 2

                @pl.when(jnp.logical_and(u2 >= 0, u2 < units))
                def _():
                    wait_g(b2)
                    start_s(u2, b2)

        for u in range(units - NBUF, units):
            wait_s(u % NBUF)

    return sc_kernel


_SC_KERNEL = _build_sc_kernel()


@jax.jit
def kernel(x, token_table, pos_table):
    xf = x.reshape(-1).astype(jnp.int32)
    pos_pat = jnp.tile(pos_table, (_LCM // MAXLEN, 1))
    out = _SC_KERNEL(token_table, xf, pos_pat)
    return out.reshape(BATCH, MAXLEN, EDIM)


# speedup vs baseline: 1.4061x; 1.0092x over previous
import functools
import math

import jax
import jax.numpy as jnp
from jax import lax
from jax.experimental import pallas as pl
from jax.experimental.pallas import tpu as pltpu
from jax.experimental.pallas import tpu_sc as plsc

VOCAB = 1000000
MAXLEN = 200
EDIM = 32
BATCH = 4096

TOTAL_ROWS = BATCH * MAXLEN
GATHER = 128
CHUNK = 2
UROWS = GATHER * CHUNK
NBUF = 8
GLAG = 5

_LCM = UROWS * MAXLEN // math.gcd(UROWS, MAXLEN)
PHASES = _LCM // UROWS


def _build_sc_kernel():
    info = plsc.get_sparse_core_info()
    nc, ns = info.num_cores, info.num_subcores
    nw = nc * ns
    rows_per_w = TOTAL_ROWS // nw
    units = rows_per_w // UROWS
    padded_steps = ((units + GLAG + NBUF - 1) // NBUF) * NBUF

    mesh = plsc.VectorSubcoreMesh(core_axis_name="c", subcore_axis_name="s")

    @functools.partial(
        pl.kernel,
        mesh=mesh,
        out_type=jax.ShapeDtypeStruct((TOTAL_ROWS, EDIM), jnp.float32),
        scratch_types=(
            [pltpu.VMEM((rows_per_w,), jnp.int32)]
            + [pltpu.VMEM((UROWS, EDIM), jnp.float32) for _ in range(NBUF)]
            + [pltpu.SemaphoreType.DMA for _ in range(3 * NBUF)]
        ),
        compiler_params=pltpu.CompilerParams(use_tc_tiling_on_sc=False),
    )
    def sc_kernel(table_hbm, idx_hbm, pos_hbm, out_hbm, idx_v, *rest):
        bufs = rest[:NBUF]
        psem = rest[NBUF:2 * NBUF]
        gsem = rest[2 * NBUF:3 * NBUF]
        ssem = rest[3 * NBUF:4 * NBUF]

        wid = lax.axis_index("s") * nc + lax.axis_index("c")
        base = wid * rows_per_w
        pltpu.sync_copy(idx_hbm.at[pl.ds(base, rows_per_w)], idx_v)

        def start_p(u, b):
            ph = lax.rem(u, PHASES)
            pltpu.async_copy(pos_hbm.at[pl.ds(ph * UROWS, UROWS)], bufs[b], psem[b])

        def wait_p(b):
            pltpu.make_async_copy(
                pos_hbm.at[pl.ds(0, UROWS)], bufs[b], psem[b]).wait()

        def start_g(u, b):
            for j in range(CHUNK):
                pltpu.async_copy(
                    table_hbm.at[idx_v.at[pl.ds(u * UROWS + j * GATHER, GATHER)]],
                    bufs[b].at[pl.ds(j * GATHER, GATHER)], gsem[b], add=True)

        def wait_g(b):
            for j in range(CHUNK):
                pltpu.make_async_copy(
                    table_hbm.at[idx_v.at[pl.ds(j * GATHER, GATHER)]],
                    bufs[b].at[pl.ds(j * GATHER, GATHER)], gsem[b]).wait()

        def start_s(u, b):
            pltpu.async_copy(
                bufs[b], out_hbm.at[pl.ds(base + u * UROWS, UROWS)], ssem[b])

        def wait_s(b):
            pltpu.make_async_copy(
                bufs[b], out_hbm.at[pl.ds(base, UROWS)], ssem[b]).wait()

        @pl.loop(0, padded_steps, step=NBUF)
        def step_loop(s):
            for b in range(NBUF):
                u = s + b

                @pl.when(jnp.logical_and(u >= NBUF, u < units))
                def _():
                    wait_s(b)

                @pl.when(u < units)
                def _():
                    start_p(u, b)

                b1 = (b - 1) % NBUF
                u1 = u - 1

                @pl.when(jnp.logical_and(u1 >= 0, u1 < units))
                def _():
                    wait_p(b1)
                    start_g(u1, b1)

                b2 = (b - GLAG) % NBUF
                u2 = u - GLAG

                @pl.when(jnp.logical_and(u2 >= 0, u2 < units))
                def _():
                    wait_g(b2)
                    start_s(u2, b2)

        for u in range(units - NBUF, units):
            wait_s(u % NBUF)

    return sc_kernel


_SC_KERNEL = _build_sc_kernel()


@jax.jit
def kernel(x, token_table, pos_table):
    xf = x.reshape(-1).astype(jnp.int32)
    pos_pat = jnp.tile(pos_table, (_LCM // MAXLEN, 1))
    out = _SC_KERNEL(token_table, xf, pos_pat)
    return out.reshape(BATCH, MAXLEN, EDIM)
